# baseline (speedup 1.0000x reference)
import functools

import jax
import jax.numpy as jnp
from jax.experimental import pallas as pl

TOP_K = 8


def _router_body(x_ref, w_ref, idx_ref, wgt_ref):
    x = x_ref[...]
    w = w_ref[...]
    logits = jax.lax.dot_general(
        w, x, (((1,), (1,)), ((), ())), preferred_element_type=jnp.float32
    )
    e_num = logits.shape[0]
    eid = jax.lax.broadcasted_iota(jnp.int32, logits.shape, 0)
    work = logits
    vals, idxs = [], []
    for _ in range(TOP_K):
        mk = jnp.max(work, axis=0, keepdims=True)
        ik = jnp.min(jnp.where(work == mk, eid, e_num), axis=0, keepdims=True)
        vals.append(mk)
        idxs.append(ik)
        work = jnp.where(eid == ik, -jnp.inf, work)
    v = jnp.concatenate(vals, axis=0)
    i = jnp.concatenate(idxs, axis=0)
    ex = jnp.exp(v - v[0:1])
    wgt = ex / jnp.sum(ex, axis=0, keepdims=True)
    idx_ref[...] = i.T
    wgt_ref[...] = wgt.T


@functools.partial(jax.jit, static_argnames=("block_t",))
def _route(flat_tokens, weight, block_t=1024):
    n, h = flat_tokens.shape
    e_num = weight.shape[0]
    grid = (n // block_t,)
    idx, wgt = pl.pallas_call(
        _router_body,
        grid=grid,
        in_specs=[
            pl.BlockSpec((block_t, h), lambda i: (i, 0)),
            pl.BlockSpec((e_num, h), lambda i: (0, 0)),
        ],
        out_specs=[
            pl.BlockSpec((block_t, TOP_K), lambda i: (i, 0)),
            pl.BlockSpec((block_t, TOP_K), lambda i: (i, 0)),
        ],
        out_shape=[
            jax.ShapeDtypeStruct((n, TOP_K), jnp.int32),
            jax.ShapeDtypeStruct((n, TOP_K), jnp.float32),
        ],
    )(flat_tokens, weight)
    return idx, wgt


def kernel(hidden_states, weight):
    if hidden_states.ndim == 2:
        hidden_states = hidden_states[:, None, :]
    bsz, seq_len, hd = hidden_states.shape
    flat = hidden_states.reshape(-1, hd)
    return _route(flat, weight, block_t=512)


# speedup vs baseline: 1.0351x; 1.0351x over previous
import functools

import jax
import jax.numpy as jnp
from jax.experimental import pallas as pl
from jax.experimental.pallas import tpu as pltpu

TOP_K = 8


def _router_body(x_ref, w_ref, idx_ref, wgt_ref):
    x = x_ref[...]
    w = w_ref[...]
    logits = jax.lax.dot_general(
        w, x, (((1,), (1,)), ((), ())), preferred_element_type=jnp.float32
    )
    e_num = logits.shape[0]
    eid = jax.lax.broadcasted_iota(jnp.int32, logits.shape, 0)
    work = logits
    vals, idxs = [], []
    for _ in range(TOP_K):
        mk = jnp.max(work, axis=0, keepdims=True)
        ik = jnp.min(jnp.where(work == mk, eid, e_num), axis=0, keepdims=True)
        vals.append(mk)
        idxs.append(ik)
        work = jnp.where(eid == ik, -jnp.inf, work)
    v = jnp.concatenate(vals, axis=0)
    i = jnp.concatenate(idxs, axis=0)
    ex = jnp.exp(v - v[0:1])
    wgt = ex / jnp.sum(ex, axis=0, keepdims=True)
    idx_ref[...] = i.T
    wgt_ref[...] = wgt.T


@functools.partial(jax.jit, static_argnames=("block_t",))
def _route(flat_tokens, weight, block_t=1024):
    n, h = flat_tokens.shape
    e_num = weight.shape[0]
    grid = (n // block_t,)
    idx, wgt = pl.pallas_call(
        _router_body,
        grid=grid,
        in_specs=[
            pl.BlockSpec((block_t, h), lambda i: (i, 0)),
            pl.BlockSpec((e_num, h), lambda i: (0, 0)),
        ],
        out_specs=[
            pl.BlockSpec((block_t, TOP_K), lambda i: (i, 0)),
            pl.BlockSpec((block_t, TOP_K), lambda i: (i, 0)),
        ],
        out_shape=[
            jax.ShapeDtypeStruct((n, TOP_K), jnp.int32),
            jax.ShapeDtypeStruct((n, TOP_K), jnp.float32),
        ],
        compiler_params=pltpu.CompilerParams(
            dimension_semantics=("parallel",),
        ),
    )(flat_tokens, weight)
    return idx, wgt


def kernel(hidden_states, weight):
    if hidden_states.ndim == 2:
        hidden_states = hidden_states[:, None, :]
    bsz, seq_len, hd = hidden_states.shape
    flat = hidden_states.reshape(-1, hd)
    return _route(flat, weight, block_t=1024)
